# baseline (speedup 1.0000x reference)
import functools

import jax
import jax.numpy as jnp
from jax import lax
from jax.experimental import pallas as pl
from jax.experimental.pallas import tpu as pltpu

NUM_VARS = 512
HIDDEN = 16
TOPK = 8
TEMP = 1.0


def _compute_s(ve_ref, wq_ref, bq_ref, wk_ref, bk_ref):
    ve = ve_ref[0]
    q = lax.dot_general(ve, wq_ref[...], (((1,), (1,)), ((), ())),
                        preferred_element_type=jnp.float32) + bq_ref[...][None, :]
    k = lax.dot_general(ve, wk_ref[...], (((1,), (1,)), ((), ())),
                        preferred_element_type=jnp.float32) + bk_ref[...][None, :]
    sim = lax.dot_general(q, k, (((1,), (1,)), ((), ())),
                          preferred_element_type=jnp.float32)
    n = sim.shape[0]
    row = lax.broadcasted_iota(jnp.int32, (n, n), 0)
    col = lax.broadcasted_iota(jnp.int32, (n, n), 1)
    sim = jnp.where(row == col, jnp.float32(-1e9), sim)

    cur = sim
    s_acc = jnp.zeros_like(sim)
    denom = jnp.zeros((n, 1), jnp.float32)
    m0 = None
    for step in range(TOPK):
        m = jnp.max(cur, axis=1, keepdims=True)
        if step == 0:
            m0 = m
        first_col = jnp.min(jnp.where(cur == m, col, n), axis=1, keepdims=True)
        sel = col == first_col
        w = jnp.exp((m - m0) * jnp.float32(1.0 / TEMP))
        s_acc = jnp.where(sel, w, s_acc)
        denom = denom + w
        cur = jnp.where(sel, jnp.float32(-3e38), cur)
    return s_acc / denom


def _fused_kernel(ve_ref, wq_ref, bq_ref, wk_ref, bk_ref, x_ref, o_ref, s_scr):
    first = (pl.program_id(0) == 0) & (pl.program_id(1) == 0)

    @pl.when(first)
    def _():
        s_scr[...] = _compute_s(ve_ref, wq_ref, bq_ref, wk_ref, bk_ref)

    o_ref[0] = lax.dot_general(
        x_ref[0], s_scr[...], (((1,), (1,)), ((), ())),
        preferred_element_type=jnp.float32)


@jax.jit
def kernel(x, var_embed, Wq, bq, Wk, bk):
    Bsz, L, N = x.shape
    BL = 4096
    out = pl.pallas_call(
        _fused_kernel,
        grid=(Bsz, L // BL),
        in_specs=[
            pl.BlockSpec((1, N, HIDDEN), lambda b, i: (0, 0, 0)),
            pl.BlockSpec((HIDDEN, HIDDEN), lambda b, i: (0, 0)),
            pl.BlockSpec((HIDDEN,), lambda b, i: (0,)),
            pl.BlockSpec((HIDDEN, HIDDEN), lambda b, i: (0, 0)),
            pl.BlockSpec((HIDDEN,), lambda b, i: (0,)),
            pl.BlockSpec((1, BL, N), lambda b, i: (b, i, 0)),
        ],
        out_specs=pl.BlockSpec((1, BL, N), lambda b, i: (b, i, 0)),
        out_shape=jax.ShapeDtypeStruct((Bsz, L, N), jnp.float32),
        scratch_shapes=[pltpu.VMEM((N, N), jnp.float32)],
    )(var_embed, Wq, bq, Wk, bk, x)
    return out


# speedup vs baseline: 1.0031x; 1.0031x over previous
import functools

import jax
import jax.numpy as jnp
from jax import lax
from jax.experimental import pallas as pl
from jax.experimental.pallas import tpu as pltpu

NUM_VARS = 512
HIDDEN = 16
TOPK = 8
TEMP = 1.0


def _compute_s(ve_ref, wq_ref, bq_ref, wk_ref, bk_ref):
    ve = ve_ref[0]
    q = lax.dot_general(ve, wq_ref[...], (((1,), (1,)), ((), ())),
                        preferred_element_type=jnp.float32) + bq_ref[...][None, :]
    k = lax.dot_general(ve, wk_ref[...], (((1,), (1,)), ((), ())),
                        preferred_element_type=jnp.float32) + bk_ref[...][None, :]
    sim = lax.dot_general(q, k, (((1,), (1,)), ((), ())),
                          preferred_element_type=jnp.float32)
    n = sim.shape[0]
    row = lax.broadcasted_iota(jnp.int32, (n, n), 0)
    col = lax.broadcasted_iota(jnp.int32, (n, n), 1)
    sim = jnp.where(row == col, jnp.float32(-1e9), sim)

    cur = sim
    s_acc = jnp.zeros_like(sim)
    denom = jnp.zeros((n, 1), jnp.float32)
    m0 = None
    for step in range(TOPK):
        m = jnp.max(cur, axis=1, keepdims=True)
        if step == 0:
            m0 = m
        first_col = jnp.min(jnp.where(cur == m, col, n), axis=1, keepdims=True)
        sel = col == first_col
        w = jnp.exp((m - m0) * jnp.float32(1.0 / TEMP))
        s_acc = jnp.where(sel, w, s_acc)
        denom = denom + w
        cur = jnp.where(sel, jnp.float32(-3e38), cur)
    return s_acc / denom


def _fused_kernel(ve_ref, wq_ref, bq_ref, wk_ref, bk_ref, x_ref, o_ref, s_scr):
    first = (pl.program_id(0) == 0) & (pl.program_id(1) == 0)

    @pl.when(first)
    def _():
        s_scr[...] = _compute_s(ve_ref, wq_ref, bq_ref, wk_ref, bk_ref)

    o_ref[0] = lax.dot_general(
        x_ref[0].astype(jnp.bfloat16), s_scr[...].astype(jnp.bfloat16),
        (((1,), (1,)), ((), ())),
        preferred_element_type=jnp.float32)


@jax.jit
def kernel(x, var_embed, Wq, bq, Wk, bk):
    Bsz, L, N = x.shape
    BL = 4096
    out = pl.pallas_call(
        _fused_kernel,
        grid=(Bsz, L // BL),
        in_specs=[
            pl.BlockSpec((1, N, HIDDEN), lambda b, i: (0, 0, 0)),
            pl.BlockSpec((HIDDEN, HIDDEN), lambda b, i: (0, 0)),
            pl.BlockSpec((HIDDEN,), lambda b, i: (0,)),
            pl.BlockSpec((HIDDEN, HIDDEN), lambda b, i: (0, 0)),
            pl.BlockSpec((HIDDEN,), lambda b, i: (0,)),
            pl.BlockSpec((1, BL, N), lambda b, i: (b, i, 0)),
        ],
        out_specs=pl.BlockSpec((1, BL, N), lambda b, i: (b, i, 0)),
        out_shape=jax.ShapeDtypeStruct((Bsz, L, N), jnp.float32),
        scratch_shapes=[pltpu.VMEM((N, N), jnp.float32)],
    )(var_embed, Wq, bq, Wk, bk, x)
    return out


# speedup vs baseline: 1.1154x; 1.1120x over previous
import functools

import jax
import jax.numpy as jnp
from jax import lax
from jax.experimental import pallas as pl
from jax.experimental.pallas import tpu as pltpu

NUM_VARS = 512
HIDDEN = 16
TOPK = 8
TEMP = 1.0


def _compute_s(vet_ref, wq_ref, bq_ref, wk_ref, bk_ref):
    vet = vet_ref[0]
    qt = lax.dot_general(wq_ref[...], vet, (((1,), (0,)), ((), ())),
                         preferred_element_type=jnp.float32) + bq_ref[...][:, None]
    kt = lax.dot_general(wk_ref[...], vet, (((1,), (0,)), ((), ())),
                         preferred_element_type=jnp.float32) + bk_ref[...][:, None]
    sim = lax.dot_general(qt, kt, (((0,), (0,)), ((), ())),
                          preferred_element_type=jnp.float32)
    n = sim.shape[0]
    row = lax.broadcasted_iota(jnp.int32, (n, n), 0)
    col = lax.broadcasted_iota(jnp.int32, (n, n), 1)
    sim = jnp.where(row == col, jnp.float32(-1e9), sim)

    cur = sim
    s_acc = jnp.zeros_like(sim)
    denom = jnp.zeros((n, 1), jnp.float32)
    m0 = None
    for step in range(TOPK):
        m = jnp.max(cur, axis=1, keepdims=True)
        if step == 0:
            m0 = m
        first_col = jnp.min(jnp.where(cur == m, col, n), axis=1, keepdims=True)
        sel = col == first_col
        w = jnp.exp((m - m0) * jnp.float32(1.0 / TEMP))
        s_acc = jnp.where(sel, w, s_acc)
        denom = denom + w
        cur = jnp.where(sel, jnp.float32(-3e38), cur)
    return s_acc / denom


def _fused_kernel(ve_ref, wq_ref, bq_ref, wk_ref, bk_ref, x_ref, o_ref, s_scr):
    first = (pl.program_id(0) == 0) & (pl.program_id(1) == 0)

    @pl.when(first)
    def _():
        s_scr[...] = _compute_s(ve_ref, wq_ref, bq_ref, wk_ref, bk_ref)

    o_ref[0] = lax.dot_general(
        x_ref[0], s_scr[...], (((1,), (1,)), ((), ())),
        preferred_element_type=jnp.float32)


@jax.jit
def kernel(x, var_embed, Wq, bq, Wk, bk):
    Bsz, L, N = x.shape
    BL = 4096
    vet = jnp.swapaxes(var_embed, 1, 2)
    out = pl.pallas_call(
        _fused_kernel,
        grid=(Bsz, L // BL),
        in_specs=[
            pl.BlockSpec((1, HIDDEN, N), lambda b, i: (0, 0, 0)),
            pl.BlockSpec((HIDDEN, HIDDEN), lambda b, i: (0, 0)),
            pl.BlockSpec((HIDDEN,), lambda b, i: (0,)),
            pl.BlockSpec((HIDDEN, HIDDEN), lambda b, i: (0, 0)),
            pl.BlockSpec((HIDDEN,), lambda b, i: (0,)),
            pl.BlockSpec((1, BL, N), lambda b, i: (b, i, 0)),
        ],
        out_specs=pl.BlockSpec((1, BL, N), lambda b, i: (b, i, 0)),
        out_shape=jax.ShapeDtypeStruct((Bsz, L, N), jnp.float32),
        scratch_shapes=[pltpu.VMEM((N, N), jnp.float32)],
    )(vet, Wq, bq, Wk, bk, x)
    return out
